# Initial kernel scaffold; baseline (speedup 1.0000x reference)
import jax
import jax.numpy as jnp
from jax.experimental import pallas as pl


def kernel(x, edge_index, W1, b1, W2, b2, Wl, bl):
    raise NotImplementedError("write your pallas kernel here")


# speedup vs baseline: 4.0285x; 4.0285x over previous
import functools

import jax
import jax.numpy as jnp
from jax import lax
from jax.experimental import pallas as pl
from jax.experimental.pallas import tpu as pltpu
from jax.experimental.pallas import tpu_sc as plsc

N = 10000
NP = 10240
E = 320000
D = 128
NC = 2
NS = 16
NW = NC * NS
EPT = E // NW
K = 80
NB = EPT // K
RPT = NP // NS
ZR = 128
_mesh = plsc.VectorSubcoreMesh(core_axis_name="c", subcore_axis_name="s")


@functools.partial(
    pl.kernel,
    out_type=jax.ShapeDtypeStruct((NC, NP, D), jnp.float32),
    mesh=_mesh,
    scratch_types=[
        pltpu.VMEM((K,), jnp.int32),
        pltpu.VMEM((K, D), jnp.float32),
        pltpu.VMEM((ZR, D), jnp.float32),
        pltpu.VMEM_SHARED((NP, D), jnp.float32),
    ],
)
def _sc_count(idx_hbm, out_hbm, idx, ones, zbuf, acc):
    cid = lax.axis_index("c")
    sid = lax.axis_index("s")
    wid = sid * NC + cid

    def fill(i, _):
        for c8 in range(D // 16):
            zbuf[i, pl.ds(c8 * 16, 16)] = jnp.zeros((16,), jnp.float32)
        return 0

    lax.fori_loop(0, ZR, fill, 0)

    def fill1(i, _):
        for c8 in range(D // 16):
            ones[i, pl.ds(c8 * 16, 16)] = jnp.ones((16,), jnp.float32)
        return 0

    lax.fori_loop(0, K, fill1, 0)
    for t in range(RPT // ZR):
        pltpu.sync_copy(zbuf, acc.at[pl.ds(sid * RPT + t * ZR, ZR)])
    plsc.subcore_barrier()

    def body(j, _):
        base = wid * EPT + j * K
        pltpu.sync_copy(idx_hbm.at[pl.ds(base, K)], idx)
        pltpu.sync_copy(ones, acc.at[idx], add=True)
        return 0

    lax.fori_loop(0, NB, body, 0)
    plsc.subcore_barrier()
    for t in range(RPT // ZR):
        r0 = sid * RPT + t * ZR
        pltpu.sync_copy(acc.at[pl.ds(r0, ZR)], out_hbm.at[cid, pl.ds(r0, ZR)])


@functools.partial(
    pl.kernel,
    out_type=jax.ShapeDtypeStruct((NC, NP, D), jnp.float32),
    mesh=_mesh,
    scratch_types=[
        pltpu.VMEM((K,), jnp.int32),
        pltpu.VMEM((K,), jnp.int32),
        pltpu.VMEM((K, D), jnp.float32),
        pltpu.VMEM((ZR, D), jnp.float32),
        pltpu.VMEM_SHARED((NP, D), jnp.float32),
        pltpu.SemaphoreType.DMA,
    ],
)
def _sc_aggregate(z_hbm, src_hbm, dst_hbm, out_hbm, sidx, didx, rows, zbuf, acc, sem):
    cid = lax.axis_index("c")
    sid = lax.axis_index("s")
    wid = sid * NC + cid

    def fill(i, _):
        for c8 in range(D // 16):
            zbuf[i, pl.ds(c8 * 16, 16)] = jnp.zeros((16,), jnp.float32)
        return 0

    lax.fori_loop(0, ZR, fill, 0)
    for t in range(RPT // ZR):
        pltpu.sync_copy(zbuf, acc.at[pl.ds(sid * RPT + t * ZR, ZR)])
    plsc.subcore_barrier()

    def body(j, _):
        base = wid * EPT + j * K
        pltpu.sync_copy(src_hbm.at[pl.ds(base, K)], sidx)
        pltpu.sync_copy(dst_hbm.at[pl.ds(base, K)], didx)
        pltpu.async_copy(z_hbm.at[sidx], rows, sem).wait()
        pltpu.sync_copy(rows, acc.at[didx], add=True)
        return 0

    lax.fori_loop(0, NB, body, 0)
    plsc.subcore_barrier()
    for t in range(RPT // ZR):
        r0 = sid * RPT + t * ZR
        pltpu.sync_copy(acc.at[pl.ds(r0, ZR)], out_hbm.at[cid, pl.ds(r0, ZR)])


R = 1000


def _deg_col(p_ref):
    col = p_ref[0, :, 0:1] + p_ref[1, :, 0:1]
    return lax.rsqrt(jnp.maximum(col, 1.0))


def _tc_scale_body(x_ref, odp_ref, o_ref):
    o_ref[...] = x_ref[...] * _deg_col(odp_ref)


_tc_scale = pl.pallas_call(
    _tc_scale_body,
    grid=(N // R,),
    in_specs=[
        pl.BlockSpec((R, D), lambda i: (i, 0)),
        pl.BlockSpec((NC, R, 1), lambda i: (0, i, 0)),
    ],
    out_specs=pl.BlockSpec((R, D), lambda i: (i, 0)),
    out_shape=jax.ShapeDtypeStruct((N, D), jnp.float32),
)


def _tc_layer1_body(a_ref, idp_ref, odp_ref, w1_ref, b1_ref, w2_ref, o_ref):
    a = (a_ref[0] + a_ref[1]) * _deg_col(idp_ref)
    h = jnp.dot(a, w1_ref[...], preferred_element_type=jnp.float32) + b1_ref[...]
    h = jnp.maximum(h, 0.0) * _deg_col(odp_ref)
    o_ref[...] = jnp.dot(h, w2_ref[...], preferred_element_type=jnp.float32)


_tc_layer1 = pl.pallas_call(
    _tc_layer1_body,
    grid=(N // R,),
    in_specs=[
        pl.BlockSpec((NC, R, D), lambda i: (0, i, 0)),
        pl.BlockSpec((NC, R, 1), lambda i: (0, i, 0)),
        pl.BlockSpec((NC, R, 1), lambda i: (0, i, 0)),
        pl.BlockSpec((128, 256), lambda i: (0, 0)),
        pl.BlockSpec((1, 256), lambda i: (0, 0)),
        pl.BlockSpec((256, 128), lambda i: (0, 0)),
    ],
    out_specs=pl.BlockSpec((R, D), lambda i: (i, 0)),
    out_shape=jax.ShapeDtypeStruct((N, D), jnp.float32),
)


def _tc_out_body(a_ref, idp_ref, b2_ref, wl_ref, bl_ref, o_ref):
    h = (a_ref[0] + a_ref[1]) * _deg_col(idp_ref) + b2_ref[...]
    h = jnp.maximum(h, 0.0)
    logits = jnp.dot(h, wl_ref[...], preferred_element_type=jnp.float32) + bl_ref[...]
    m = jnp.max(logits, axis=-1, keepdims=True)
    e = jnp.exp(logits - m)
    o_ref[...] = e / jnp.sum(e, axis=-1, keepdims=True)


_tc_out = pl.pallas_call(
    _tc_out_body,
    grid=(N // R,),
    in_specs=[
        pl.BlockSpec((NC, R, D), lambda i: (0, i, 0)),
        pl.BlockSpec((NC, R, 1), lambda i: (0, i, 0)),
        pl.BlockSpec((1, 128), lambda i: (0, 0)),
        pl.BlockSpec((128, 40), lambda i: (0, 0)),
        pl.BlockSpec((1, 40), lambda i: (0, 0)),
    ],
    out_specs=pl.BlockSpec((R, 40), lambda i: (i, 0)),
    out_shape=jax.ShapeDtypeStruct((N, 40), jnp.float32),
)


def kernel(x, edge_index, W1, b1, W2, b2, Wl, bl):
    src = edge_index[0]
    dst = edge_index[1]
    odp = _sc_count(src)[:, :, 0:1]
    idp = _sc_count(dst)[:, :, 0:1]
    x1 = _tc_scale(x, odp)
    a1p = _sc_aggregate(x1, src, dst)
    t2 = _tc_layer1(a1p, idp, odp, W1, b1.reshape(1, -1), W2)
    a2p = _sc_aggregate(t2, src, dst)
    return _tc_out(a2p, idp, b2.reshape(1, -1), Wl, bl.reshape(1, -1))


# speedup vs baseline: 7.1854x; 1.7836x over previous
import functools

import jax
import jax.numpy as jnp
from jax import lax
from jax.experimental import pallas as pl
from jax.experimental.pallas import tpu as pltpu
from jax.experimental.pallas import tpu_sc as plsc

N = 10000
NP = 10240
E = 320000
D = 128
B = 128
ER = E // B
SR = 8
SLOTS = ER // SR
TAILR = ER - SLOTS * SR
NC = 2
NS = 16
NW = NC * NS
RPT = NP // NS
ZR = 128

_mesh = plsc.VectorSubcoreMesh(core_axis_name="c", subcore_axis_name="s")


def _zero_fill(buf, nrows):
    def fill(i, _):
        for c8 in range(D // 16):
            buf[i, pl.ds(c8 * 16, 16)] = jnp.zeros((16,), jnp.float32)
        return 0

    lax.fori_loop(0, nrows, fill, 0)


@functools.partial(
    pl.kernel,
    out_type=jax.ShapeDtypeStruct((NC, NP, D), jnp.float32),
    mesh=_mesh,
    scratch_types=[
        pltpu.VMEM((SR, B), jnp.int32),
        pltpu.VMEM((B, D), jnp.float32),
        pltpu.VMEM_SHARED((NP, D), jnp.float32),
        pltpu.SemaphoreType.DMA,
    ],
)
def _sc_count2(src2_hbm, dst2_hbm, out_hbm, idxv, ones, acc, sem):
    cid = lax.axis_index("c")
    sid = lax.axis_index("s")

    _zero_fill(ones, B)
    for t in range(RPT // ZR):
        pltpu.sync_copy(ones, acc.at[pl.ds(sid * RPT + t * ZR, ZR)])

    def fill1(i, _):
        for c8 in range(D // 16):
            ones[i, pl.ds(c8 * 16, 16)] = jnp.ones((16,), jnp.float32)
        return 0

    lax.fori_loop(0, B, fill1, 0)
    plsc.subcore_barrier()

    gcnt = (SLOTS + NS - 1 - sid) // NS

    def make_body(edges2):
        def body(s, _):
            @pl.when(s < gcnt)
            def _():
                brow = (sid + NS * s) * SR
                pltpu.sync_copy(edges2.at[pl.ds(brow, SR)], idxv)
                hs = []
                for r in range(SR):
                    hs.append(pltpu.async_copy(ones, acc.at[idxv.at[r]], sem, add=True))
                for h in hs:
                    h.wait()
            return 0

        return body

    def tail(edges2):
        pltpu.sync_copy(edges2.at[pl.ds(SLOTS * SR, TAILR)], idxv.at[pl.ds(0, TAILR)])
        for r in range(TAILR):
            pltpu.sync_copy(ones, acc.at[idxv.at[r]], add=True)

    @pl.when(cid == 0)
    def _():
        lax.fori_loop(0, (SLOTS + NS - 1) // NS, make_body(src2_hbm), 0)

        @pl.when(sid == 0)
        def _():
            tail(src2_hbm)

    @pl.when(cid == 1)
    def _():
        lax.fori_loop(0, (SLOTS + NS - 1) // NS, make_body(dst2_hbm), 0)

        @pl.when(sid == 0)
        def _():
            tail(dst2_hbm)

    plsc.subcore_barrier()
    for t in range(RPT // ZR):
        r0 = sid * RPT + t * ZR
        pltpu.sync_copy(acc.at[pl.ds(r0, ZR)], out_hbm.at[cid, pl.ds(r0, ZR)])


@functools.partial(
    pl.kernel,
    out_type=jax.ShapeDtypeStruct((NC, NP, D), jnp.float32),
    mesh=_mesh,
    scratch_types=[
        pltpu.VMEM((SR, B), jnp.int32),
        pltpu.VMEM((SR, B), jnp.int32),
        pltpu.VMEM((B, D), jnp.float32),
        pltpu.VMEM((B, D), jnp.float32),
        pltpu.VMEM_SHARED((NP, D), jnp.float32),
        pltpu.SemaphoreType.DMA,
        pltpu.SemaphoreType.DMA,
    ],
)
def _sc_aggregate(z_hbm, src2_hbm, dst2_hbm, out_hbm,
                  sidxv, didxv, rows0, rows1, acc, gsem0, gsem1):
    cid = lax.axis_index("c")
    sid = lax.axis_index("s")
    wid = sid * NC + cid

    _zero_fill(rows0, B)
    for t in range(RPT // ZR):
        pltpu.sync_copy(rows0, acc.at[pl.ds(sid * RPT + t * ZR, ZR)])
    plsc.subcore_barrier()

    rows = (rows0, rows1)
    gsems = (gsem0, gsem1)
    gcnt = (SLOTS + NW - 1 - wid) // NW

    def body(s, _):
        @pl.when(s < gcnt)
        def _():
            brow = (wid + NW * s) * SR
            pltpu.sync_copy(src2_hbm.at[pl.ds(brow, SR)], sidxv)
            pltpu.sync_copy(dst2_hbm.at[pl.ds(brow, SR)], didxv)
            hs = [None, None]
            hs[0] = pltpu.async_copy(z_hbm.at[sidxv.at[0]], rows[0], gsems[0])
            for r in range(SR):
                p = r % 2
                hs[p].wait()
                if r + 1 < SR:
                    hs[1 - p] = pltpu.async_copy(
                        z_hbm.at[sidxv.at[r + 1]], rows[1 - p], gsems[1 - p])
                pltpu.sync_copy(rows[p], acc.at[didxv.at[r]], add=True)
        return 0

    lax.fori_loop(0, (SLOTS + NW - 1) // NW, body, 0)

    @pl.when(wid == 0)
    def _():
        pltpu.sync_copy(src2_hbm.at[pl.ds(SLOTS * SR, TAILR)], sidxv.at[pl.ds(0, TAILR)])
        pltpu.sync_copy(dst2_hbm.at[pl.ds(SLOTS * SR, TAILR)], didxv.at[pl.ds(0, TAILR)])
        for r in range(TAILR):
            pltpu.async_copy(z_hbm.at[sidxv.at[r]], rows0, gsem0).wait()
            pltpu.sync_copy(rows0, acc.at[didxv.at[r]], add=True)

    plsc.subcore_barrier()
    for t in range(RPT // ZR):
        r0 = sid * RPT + t * ZR
        pltpu.sync_copy(acc.at[pl.ds(r0, ZR)], out_hbm.at[cid, pl.ds(r0, ZR)])


R = 1000


def _deg_col(p_ref):
    return lax.rsqrt(jnp.maximum(p_ref[0, :, 0:1], 1.0))


def _tc_scale_body(x_ref, odp_ref, o_ref):
    o_ref[...] = x_ref[...] * _deg_col(odp_ref)


_tc_scale = pl.pallas_call(
    _tc_scale_body,
    grid=(N // R,),
    in_specs=[
        pl.BlockSpec((R, D), lambda i: (i, 0)),
        pl.BlockSpec((1, R, 1), lambda i: (0, i, 0)),
    ],
    out_specs=pl.BlockSpec((R, D), lambda i: (i, 0)),
    out_shape=jax.ShapeDtypeStruct((N, D), jnp.float32),
)


def _tc_layer1_body(a_ref, idp_ref, odp_ref, w1_ref, b1_ref, w2_ref, o_ref):
    a = (a_ref[0] + a_ref[1]) * _deg_col(idp_ref)
    h = jnp.dot(a, w1_ref[...], preferred_element_type=jnp.float32) + b1_ref[...]
    h = jnp.maximum(h, 0.0) * _deg_col(odp_ref)
    o_ref[...] = jnp.dot(h, w2_ref[...], preferred_element_type=jnp.float32)


_tc_layer1 = pl.pallas_call(
    _tc_layer1_body,
    grid=(N // R,),
    in_specs=[
        pl.BlockSpec((NC, R, D), lambda i: (0, i, 0)),
        pl.BlockSpec((1, R, 1), lambda i: (0, i, 0)),
        pl.BlockSpec((1, R, 1), lambda i: (0, i, 0)),
        pl.BlockSpec((128, 256), lambda i: (0, 0)),
        pl.BlockSpec((1, 256), lambda i: (0, 0)),
        pl.BlockSpec((256, 128), lambda i: (0, 0)),
    ],
    out_specs=pl.BlockSpec((R, D), lambda i: (i, 0)),
    out_shape=jax.ShapeDtypeStruct((N, D), jnp.float32),
)


def _tc_out_body(a_ref, idp_ref, b2_ref, wl_ref, bl_ref, o_ref):
    h = (a_ref[0] + a_ref[1]) * _deg_col(idp_ref) + b2_ref[...]
    h = jnp.maximum(h, 0.0)
    logits = jnp.dot(h, wl_ref[...], preferred_element_type=jnp.float32) + bl_ref[...]
    m = jnp.max(logits, axis=-1, keepdims=True)
    e = jnp.exp(logits - m)
    o_ref[...] = e / jnp.sum(e, axis=-1, keepdims=True)


_tc_out = pl.pallas_call(
    _tc_out_body,
    grid=(N // R,),
    in_specs=[
        pl.BlockSpec((NC, R, D), lambda i: (0, i, 0)),
        pl.BlockSpec((1, R, 1), lambda i: (0, i, 0)),
        pl.BlockSpec((1, 128), lambda i: (0, 0)),
        pl.BlockSpec((128, 40), lambda i: (0, 0)),
        pl.BlockSpec((1, 40), lambda i: (0, 0)),
    ],
    out_specs=pl.BlockSpec((R, 40), lambda i: (i, 0)),
    out_shape=jax.ShapeDtypeStruct((N, 40), jnp.float32),
)


def kernel(x, edge_index, W1, b1, W2, b2, Wl, bl):
    src2 = edge_index[0].reshape(ER, B)
    dst2 = edge_index[1].reshape(ER, B)
    deg = _sc_count2(src2, dst2)
    odp = deg[0:1, :, 0:1]
    idp = deg[1:2, :, 0:1]
    x1 = _tc_scale(x, odp)
    a1p = _sc_aggregate(x1, src2, dst2)
    t2 = _tc_layer1(a1p, idp, odp, W1, b1.reshape(1, -1), W2)
    a2p = _sc_aggregate(t2, src2, dst2)
    return _tc_out(a2p, idp, b2.reshape(1, -1), Wl, bl.reshape(1, -1))


# speedup vs baseline: 7.4112x; 1.0314x over previous
import functools

import jax
import jax.numpy as jnp
from jax import lax
from jax.experimental import pallas as pl
from jax.experimental.pallas import tpu as pltpu
from jax.experimental.pallas import tpu_sc as plsc

N = 10000
NP = 10240
E = 320000
D = 128
B = 128
ER = E // B
SR = 8
SLOTS = ER // SR
TAILR = ER - SLOTS * SR
NC = 2
NS = 16
NW = NC * NS
RPT = NP // NS
ZR = 128

_mesh = plsc.VectorSubcoreMesh(core_axis_name="c", subcore_axis_name="s")


def _zero_fill(buf, nrows):
    def fill(i, _):
        for c8 in range(D // 16):
            buf[i, pl.ds(c8 * 16, 16)] = jnp.zeros((16,), jnp.float32)
        return 0

    lax.fori_loop(0, nrows, fill, 0)


@functools.partial(
    pl.kernel,
    out_type=jax.ShapeDtypeStruct((NC, NP, D), jnp.float32),
    mesh=_mesh,
    scratch_types=[
        pltpu.VMEM((SR, B), jnp.int32),
        pltpu.VMEM((B, D), jnp.float32),
        pltpu.VMEM_SHARED((NP, D), jnp.float32),
        pltpu.SemaphoreType.DMA,
    ],
)
def _sc_count2(src2_hbm, dst2_hbm, out_hbm, idxv, ones, acc, sem):
    cid = lax.axis_index("c")
    sid = lax.axis_index("s")

    _zero_fill(ones, B)
    for t in range(RPT // ZR):
        pltpu.sync_copy(ones, acc.at[pl.ds(sid * RPT + t * ZR, ZR)])

    def fill1(i, _):
        for c8 in range(D // 16):
            ones[i, pl.ds(c8 * 16, 16)] = jnp.ones((16,), jnp.float32)
        return 0

    lax.fori_loop(0, B, fill1, 0)
    plsc.subcore_barrier()

    gcnt = (SLOTS + NS - 1 - sid) // NS

    def make_body(edges2):
        def body(s, _):
            @pl.when(s < gcnt)
            def _():
                brow = (sid + NS * s) * SR
                pltpu.sync_copy(edges2.at[pl.ds(brow, SR)], idxv)
                hs = []
                for r in range(SR):
                    hs.append(pltpu.async_copy(ones, acc.at[idxv.at[r]], sem, add=True))
                for h in hs:
                    h.wait()
            return 0

        return body

    def tail(edges2):
        pltpu.sync_copy(edges2.at[pl.ds(SLOTS * SR, TAILR)], idxv.at[pl.ds(0, TAILR)])
        for r in range(TAILR):
            pltpu.sync_copy(ones, acc.at[idxv.at[r]], add=True)

    @pl.when(cid == 0)
    def _():
        lax.fori_loop(0, (SLOTS + NS - 1) // NS, make_body(src2_hbm), 0)

        @pl.when(sid == 0)
        def _():
            tail(src2_hbm)

    @pl.when(cid == 1)
    def _():
        lax.fori_loop(0, (SLOTS + NS - 1) // NS, make_body(dst2_hbm), 0)

        @pl.when(sid == 0)
        def _():
            tail(dst2_hbm)

    plsc.subcore_barrier()
    for t in range(RPT // ZR):
        r0 = sid * RPT + t * ZR
        pltpu.sync_copy(acc.at[pl.ds(r0, ZR)], out_hbm.at[cid, pl.ds(r0, ZR)])


@functools.partial(
    pl.kernel,
    out_type=jax.ShapeDtypeStruct((NC, NP, D), jnp.float32),
    mesh=_mesh,
    scratch_types=[
        pltpu.VMEM((SR, B), jnp.int32),
        pltpu.VMEM((SR, B), jnp.int32),
        pltpu.VMEM((B, D), jnp.float32),
        pltpu.VMEM((B, D), jnp.float32),
        pltpu.VMEM_SHARED((NP, D), jnp.float32),
        pltpu.SemaphoreType.DMA,
        pltpu.SemaphoreType.DMA,
        pltpu.SemaphoreType.DMA,
        pltpu.SemaphoreType.DMA,
    ],
)
def _sc_aggregate(z_hbm, src2_hbm, dst2_hbm, out_hbm,
                  sidxv, didxv, rows0, rows1, acc, gsem0, gsem1, ssem0, ssem1):
    cid = lax.axis_index("c")
    sid = lax.axis_index("s")
    wid = sid * NC + cid

    _zero_fill(rows0, B)
    for t in range(RPT // ZR):
        pltpu.sync_copy(rows0, acc.at[pl.ds(sid * RPT + t * ZR, ZR)])
    plsc.subcore_barrier()

    rows = (rows0, rows1)
    gsems = (gsem0, gsem1)
    ssems = (ssem0, ssem1)
    gcnt = (SLOTS + NW - 1 - wid) // NW

    def body(s, _):
        @pl.when(s < gcnt)
        def _():
            brow = (wid + NW * s) * SR
            pltpu.sync_copy(src2_hbm.at[pl.ds(brow, SR)], sidxv)
            pltpu.sync_copy(dst2_hbm.at[pl.ds(brow, SR)], didxv)
            hg = [None, None]
            hs = [None, None]
            hg[0] = pltpu.async_copy(z_hbm.at[sidxv.at[0]], rows[0], gsems[0])
            for r in range(SR):
                p = r % 2
                hg[p].wait()
                hs[p] = pltpu.async_copy(rows[p], acc.at[didxv.at[r]], ssems[p], add=True)
                if r + 1 < SR:
                    if hs[1 - p] is not None:
                        hs[1 - p].wait()
                    hg[1 - p] = pltpu.async_copy(
                        z_hbm.at[sidxv.at[r + 1]], rows[1 - p], gsems[1 - p])
            hs[0].wait()
            hs[1].wait()
        return 0

    lax.fori_loop(0, (SLOTS + NW - 1) // NW, body, 0)

    @pl.when(wid == NW - 1)
    def _():
        pltpu.sync_copy(src2_hbm.at[pl.ds(SLOTS * SR, TAILR)], sidxv.at[pl.ds(0, TAILR)])
        pltpu.sync_copy(dst2_hbm.at[pl.ds(SLOTS * SR, TAILR)], didxv.at[pl.ds(0, TAILR)])
        for r in range(TAILR):
            pltpu.async_copy(z_hbm.at[sidxv.at[r]], rows0, gsem0).wait()
            pltpu.sync_copy(rows0, acc.at[didxv.at[r]], add=True)

    plsc.subcore_barrier()
    for t in range(RPT // ZR):
        r0 = sid * RPT + t * ZR
        pltpu.sync_copy(acc.at[pl.ds(r0, ZR)], out_hbm.at[cid, pl.ds(r0, ZR)])


R = 1000


def _deg_col(p_ref):
    return lax.rsqrt(jnp.maximum(p_ref[0, :, 0:1], 1.0))


def _tc_scale_body(x_ref, odp_ref, o_ref):
    o_ref[...] = x_ref[...] * _deg_col(odp_ref)


_tc_scale = pl.pallas_call(
    _tc_scale_body,
    grid=(N // R,),
    in_specs=[
        pl.BlockSpec((R, D), lambda i: (i, 0)),
        pl.BlockSpec((1, R, 1), lambda i: (0, i, 0)),
    ],
    out_specs=pl.BlockSpec((R, D), lambda i: (i, 0)),
    out_shape=jax.ShapeDtypeStruct((N, D), jnp.float32),
)


def _tc_layer1_body(a_ref, idp_ref, odp_ref, w1_ref, b1_ref, w2_ref, o_ref):
    a = (a_ref[0] + a_ref[1]) * _deg_col(idp_ref)
    h = jnp.dot(a, w1_ref[...], preferred_element_type=jnp.float32) + b1_ref[...]
    h = jnp.maximum(h, 0.0) * _deg_col(odp_ref)
    o_ref[...] = jnp.dot(h, w2_ref[...], preferred_element_type=jnp.float32)


_tc_layer1 = pl.pallas_call(
    _tc_layer1_body,
    grid=(N // R,),
    in_specs=[
        pl.BlockSpec((NC, R, D), lambda i: (0, i, 0)),
        pl.BlockSpec((1, R, 1), lambda i: (0, i, 0)),
        pl.BlockSpec((1, R, 1), lambda i: (0, i, 0)),
        pl.BlockSpec((128, 256), lambda i: (0, 0)),
        pl.BlockSpec((1, 256), lambda i: (0, 0)),
        pl.BlockSpec((256, 128), lambda i: (0, 0)),
    ],
    out_specs=pl.BlockSpec((R, D), lambda i: (i, 0)),
    out_shape=jax.ShapeDtypeStruct((N, D), jnp.float32),
)


def _tc_out_body(a_ref, idp_ref, b2_ref, wl_ref, bl_ref, o_ref):
    h = (a_ref[0] + a_ref[1]) * _deg_col(idp_ref) + b2_ref[...]
    h = jnp.maximum(h, 0.0)
    logits = jnp.dot(h, wl_ref[...], preferred_element_type=jnp.float32) + bl_ref[...]
    m = jnp.max(logits, axis=-1, keepdims=True)
    e = jnp.exp(logits - m)
    o_ref[...] = e / jnp.sum(e, axis=-1, keepdims=True)


_tc_out = pl.pallas_call(
    _tc_out_body,
    grid=(N // R,),
    in_specs=[
        pl.BlockSpec((NC, R, D), lambda i: (0, i, 0)),
        pl.BlockSpec((1, R, 1), lambda i: (0, i, 0)),
        pl.BlockSpec((1, 128), lambda i: (0, 0)),
        pl.BlockSpec((128, 40), lambda i: (0, 0)),
        pl.BlockSpec((1, 40), lambda i: (0, 0)),
    ],
    out_specs=pl.BlockSpec((R, 40), lambda i: (i, 0)),
    out_shape=jax.ShapeDtypeStruct((N, 40), jnp.float32),
)


def kernel(x, edge_index, W1, b1, W2, b2, Wl, bl):
    src2 = edge_index[0].reshape(ER, B)
    dst2 = edge_index[1].reshape(ER, B)
    deg = _sc_count2(src2, dst2)
    odp = deg[0:1, :, 0:1]
    idp = deg[1:2, :, 0:1]
    x1 = _tc_scale(x, odp)
    a1p = _sc_aggregate(x1, src2, dst2)
    t2 = _tc_layer1(a1p, idp, odp, W1, b1.reshape(1, -1), W2)
    a2p = _sc_aggregate(t2, src2, dst2)
    return _tc_out(a2p, idp, b2.reshape(1, -1), Wl, bl.reshape(1, -1))


# speedup vs baseline: 7.6298x; 1.0295x over previous
import functools

import jax
import jax.numpy as jnp
from jax import lax
from jax.experimental import pallas as pl
from jax.experimental.pallas import tpu as pltpu
from jax.experimental.pallas import tpu_sc as plsc

N = 10000
NP = 10240
E = 320000
D = 128
B = 128
ER = E // B
SR = 8
SLOTS = ER // SR
TAILR = ER - SLOTS * SR
CSR = 16
CSLOTS = ER // CSR
CTAILR = ER - CSLOTS * CSR
NC = 2
NS = 16
NW = NC * NS
RPT = NP // NS
ZR = 128

_mesh = plsc.VectorSubcoreMesh(core_axis_name="c", subcore_axis_name="s")


def _zero_fill(buf, nrows):
    def fill(i, _):
        for c8 in range(D // 16):
            buf[i, pl.ds(c8 * 16, 16)] = jnp.zeros((16,), jnp.float32)
        return 0

    lax.fori_loop(0, nrows, fill, 0)


@functools.partial(
    pl.kernel,
    out_type=jax.ShapeDtypeStruct((NC, NP, D), jnp.float32),
    mesh=_mesh,
    scratch_types=[
        pltpu.VMEM((CSR, B), jnp.int32),
        pltpu.VMEM((B, D), jnp.float32),
        pltpu.VMEM_SHARED((NP, D), jnp.float32),
        pltpu.SemaphoreType.DMA,
    ],
)
def _sc_count2(src2_hbm, dst2_hbm, out_hbm, idxv, ones, acc, sem):
    cid = lax.axis_index("c")
    sid = lax.axis_index("s")

    _zero_fill(ones, B)
    for t in range(RPT // ZR):
        pltpu.sync_copy(ones, acc.at[pl.ds(sid * RPT + t * ZR, ZR)])

    def fill1(i, _):
        for c8 in range(D // 16):
            ones[i, pl.ds(c8 * 16, 16)] = jnp.ones((16,), jnp.float32)
        return 0

    lax.fori_loop(0, B, fill1, 0)
    plsc.subcore_barrier()

    gcnt = (CSLOTS + NS - 1 - sid) // NS

    def make_body(edges2):
        def body(s, _):
            @pl.when(s < gcnt)
            def _():
                brow = (sid + NS * s) * CSR
                pltpu.sync_copy(edges2.at[pl.ds(brow, CSR)], idxv)
                hs = []
                for r in range(CSR):
                    hs.append(pltpu.async_copy(ones, acc.at[idxv.at[r]], sem, add=True))
                for h in hs:
                    h.wait()
            return 0

        return body

    def tail(edges2):
        pltpu.sync_copy(edges2.at[pl.ds(CSLOTS * CSR, CTAILR)], idxv.at[pl.ds(0, CTAILR)])
        for r in range(CTAILR):
            pltpu.sync_copy(ones, acc.at[idxv.at[r]], add=True)

    @pl.when(cid == 0)
    def _():
        lax.fori_loop(0, (CSLOTS + NS - 1) // NS, make_body(src2_hbm), 0)

        @pl.when(sid == NS - 1)
        def _():
            tail(src2_hbm)

    @pl.when(cid == 1)
    def _():
        lax.fori_loop(0, (CSLOTS + NS - 1) // NS, make_body(dst2_hbm), 0)

        @pl.when(sid == NS - 1)
        def _():
            tail(dst2_hbm)

    plsc.subcore_barrier()
    for t in range(RPT // ZR):
        r0 = sid * RPT + t * ZR
        pltpu.sync_copy(acc.at[pl.ds(r0, ZR)], out_hbm.at[cid, pl.ds(r0, ZR)])


@functools.partial(
    pl.kernel,
    out_type=jax.ShapeDtypeStruct((NC, NP, D), jnp.float32),
    mesh=_mesh,
    scratch_types=[
        pltpu.VMEM((SR, B), jnp.int32),
        pltpu.VMEM((SR, B), jnp.int32),
        pltpu.VMEM((B, D), jnp.float32),
        pltpu.VMEM((B, D), jnp.float32),
        pltpu.VMEM_SHARED((NP, D), jnp.float32),
        pltpu.SemaphoreType.DMA,
        pltpu.SemaphoreType.DMA,
        pltpu.SemaphoreType.DMA,
        pltpu.SemaphoreType.DMA,
    ],
)
def _sc_aggregate(z_hbm, src2_hbm, dst2_hbm, out_hbm,
                  sidxv, didxv, rows0, rows1, acc, gsem0, gsem1, ssem0, ssem1):
    cid = lax.axis_index("c")
    sid = lax.axis_index("s")
    wid = sid * NC + cid

    _zero_fill(rows0, B)
    for t in range(RPT // ZR):
        pltpu.sync_copy(rows0, acc.at[pl.ds(sid * RPT + t * ZR, ZR)])
    plsc.subcore_barrier()

    rows = (rows0, rows1)
    gsems = (gsem0, gsem1)
    ssems = (ssem0, ssem1)
    gcnt = (SLOTS + NW - 1 - wid) // NW

    def body(s, _):
        @pl.when(s < gcnt)
        def _():
            brow = (wid + NW * s) * SR
            pltpu.sync_copy(src2_hbm.at[pl.ds(brow, SR)], sidxv)
            pltpu.sync_copy(dst2_hbm.at[pl.ds(brow, SR)], didxv)
            hg = [None, None]
            hs = [None, None]
            hg[0] = pltpu.async_copy(z_hbm.at[sidxv.at[0]], rows[0], gsems[0])
            for r in range(SR):
                p = r % 2
                hg[p].wait()
                hs[p] = pltpu.async_copy(rows[p], acc.at[didxv.at[r]], ssems[p], add=True)
                if r + 1 < SR:
                    if hs[1 - p] is not None:
                        hs[1 - p].wait()
                    hg[1 - p] = pltpu.async_copy(
                        z_hbm.at[sidxv.at[r + 1]], rows[1 - p], gsems[1 - p])
            hs[0].wait()
            hs[1].wait()
        return 0

    lax.fori_loop(0, (SLOTS + NW - 1) // NW, body, 0)

    @pl.when(wid == NW - 1)
    def _():
        pltpu.sync_copy(src2_hbm.at[pl.ds(SLOTS * SR, TAILR)], sidxv.at[pl.ds(0, TAILR)])
        pltpu.sync_copy(dst2_hbm.at[pl.ds(SLOTS * SR, TAILR)], didxv.at[pl.ds(0, TAILR)])
        for r in range(TAILR):
            pltpu.async_copy(z_hbm.at[sidxv.at[r]], rows0, gsem0).wait()
            pltpu.sync_copy(rows0, acc.at[didxv.at[r]], add=True)

    plsc.subcore_barrier()
    for t in range(RPT // ZR):
        r0 = sid * RPT + t * ZR
        pltpu.sync_copy(acc.at[pl.ds(r0, ZR)], out_hbm.at[cid, pl.ds(r0, ZR)])


R = 10000


def _deg_col(p_ref):
    return lax.rsqrt(jnp.maximum(p_ref[0, :, 0:1], 1.0))


def _tc_scale_body(x_ref, odp_ref, o_ref):
    o_ref[...] = x_ref[...] * _deg_col(odp_ref)


_tc_scale = pl.pallas_call(
    _tc_scale_body,
    grid=(N // R,),
    in_specs=[
        pl.BlockSpec((R, D), lambda i: (i, 0)),
        pl.BlockSpec((1, R, 1), lambda i: (0, i, 0)),
    ],
    out_specs=pl.BlockSpec((R, D), lambda i: (i, 0)),
    out_shape=jax.ShapeDtypeStruct((N, D), jnp.float32),
)


def _tc_layer1_body(a_ref, idp_ref, odp_ref, w1_ref, b1_ref, w2_ref, o_ref):
    a = (a_ref[0] + a_ref[1]) * _deg_col(idp_ref)
    h = jnp.dot(a, w1_ref[...], preferred_element_type=jnp.float32) + b1_ref[...]
    h = jnp.maximum(h, 0.0) * _deg_col(odp_ref)
    o_ref[...] = jnp.dot(h, w2_ref[...], preferred_element_type=jnp.float32)


_tc_layer1 = pl.pallas_call(
    _tc_layer1_body,
    grid=(N // R,),
    in_specs=[
        pl.BlockSpec((NC, R, D), lambda i: (0, i, 0)),
        pl.BlockSpec((1, R, 1), lambda i: (0, i, 0)),
        pl.BlockSpec((1, R, 1), lambda i: (0, i, 0)),
        pl.BlockSpec((128, 256), lambda i: (0, 0)),
        pl.BlockSpec((1, 256), lambda i: (0, 0)),
        pl.BlockSpec((256, 128), lambda i: (0, 0)),
    ],
    out_specs=pl.BlockSpec((R, D), lambda i: (i, 0)),
    out_shape=jax.ShapeDtypeStruct((N, D), jnp.float32),
)


def _tc_out_body(a_ref, idp_ref, b2_ref, wl_ref, bl_ref, o_ref):
    h = (a_ref[0] + a_ref[1]) * _deg_col(idp_ref) + b2_ref[...]
    h = jnp.maximum(h, 0.0)
    logits = jnp.dot(h, wl_ref[...], preferred_element_type=jnp.float32) + bl_ref[...]
    m = jnp.max(logits, axis=-1, keepdims=True)
    e = jnp.exp(logits - m)
    o_ref[...] = e / jnp.sum(e, axis=-1, keepdims=True)


_tc_out = pl.pallas_call(
    _tc_out_body,
    grid=(N // R,),
    in_specs=[
        pl.BlockSpec((NC, R, D), lambda i: (0, i, 0)),
        pl.BlockSpec((1, R, 1), lambda i: (0, i, 0)),
        pl.BlockSpec((1, 128), lambda i: (0, 0)),
        pl.BlockSpec((128, 40), lambda i: (0, 0)),
        pl.BlockSpec((1, 40), lambda i: (0, 0)),
    ],
    out_specs=pl.BlockSpec((R, 40), lambda i: (i, 0)),
    out_shape=jax.ShapeDtypeStruct((N, 40), jnp.float32),
)


def kernel(x, edge_index, W1, b1, W2, b2, Wl, bl):
    src2 = edge_index[0].reshape(ER, B)
    dst2 = edge_index[1].reshape(ER, B)
    deg = _sc_count2(src2, dst2)
    odp = deg[0:1, :, 0:1]
    idp = deg[1:2, :, 0:1]
    x1 = _tc_scale(x, odp)
    a1p = _sc_aggregate(x1, src2, dst2)
    t2 = _tc_layer1(a1p, idp, odp, W1, b1.reshape(1, -1), W2)
    a2p = _sc_aggregate(t2, src2, dst2)
    return _tc_out(a2p, idp, b2.reshape(1, -1), Wl, bl.reshape(1, -1))


# speedup vs baseline: 7.7148x; 1.0111x over previous
import functools

import jax
import jax.numpy as jnp
from jax import lax
from jax.experimental import pallas as pl
from jax.experimental.pallas import tpu as pltpu
from jax.experimental.pallas import tpu_sc as plsc

N = 10000
NP = 10240
E = 320000
D = 128
B = 128
ER = E // B
SR = 8
SLOTS = ER // SR
TAILR = ER - SLOTS * SR
CSR = 16
CSLOTS = ER // CSR
CTAILR = ER - CSLOTS * CSR
NC = 2
NS = 16
NW = NC * NS
RPT = NP // NS
ZR = 128

_mesh = plsc.VectorSubcoreMesh(core_axis_name="c", subcore_axis_name="s")


def _zero_fill(buf, nrows):
    def fill(i, _):
        for c8 in range(D // 16):
            buf[i, pl.ds(c8 * 16, 16)] = jnp.zeros((16,), jnp.float32)
        return 0

    lax.fori_loop(0, nrows, fill, 0)


@functools.partial(
    pl.kernel,
    out_type=jax.ShapeDtypeStruct((NC, NP, D), jnp.float32),
    mesh=_mesh,
    scratch_types=[
        pltpu.VMEM((CSR, B), jnp.int32),
        pltpu.VMEM((B, D), jnp.float32),
        pltpu.VMEM_SHARED((NP, D), jnp.float32),
        pltpu.SemaphoreType.DMA,
    ],
)
def _sc_count2(src2_hbm, dst2_hbm, out_hbm, idxv, ones, acc, sem):
    cid = lax.axis_index("c")
    sid = lax.axis_index("s")

    _zero_fill(ones, B)
    for t in range(RPT // ZR):
        pltpu.sync_copy(ones, acc.at[pl.ds(sid * RPT + t * ZR, ZR)])

    def fill1(i, _):
        for c8 in range(D // 16):
            ones[i, pl.ds(c8 * 16, 16)] = jnp.ones((16,), jnp.float32)
        return 0

    lax.fori_loop(0, B, fill1, 0)
    plsc.subcore_barrier()

    gcnt = (CSLOTS + NS - 1 - sid) // NS

    def make_body(edges2):
        def body(s, _):
            @pl.when(s < gcnt)
            def _():
                brow = (sid + NS * s) * CSR
                pltpu.sync_copy(edges2.at[pl.ds(brow, CSR)], idxv)
                hs = []
                for r in range(CSR):
                    hs.append(pltpu.async_copy(ones, acc.at[idxv.at[r]], sem, add=True))
                for h in hs:
                    h.wait()
            return 0

        return body

    def tail(edges2):
        pltpu.sync_copy(edges2.at[pl.ds(CSLOTS * CSR, CTAILR)], idxv.at[pl.ds(0, CTAILR)])
        for r in range(CTAILR):
            pltpu.sync_copy(ones, acc.at[idxv.at[r]], add=True)

    @pl.when(cid == 0)
    def _():
        lax.fori_loop(0, (CSLOTS + NS - 1) // NS, make_body(src2_hbm), 0)

        @pl.when(sid == NS - 1)
        def _():
            tail(src2_hbm)

    @pl.when(cid == 1)
    def _():
        lax.fori_loop(0, (CSLOTS + NS - 1) // NS, make_body(dst2_hbm), 0)

        @pl.when(sid == NS - 1)
        def _():
            tail(dst2_hbm)

    plsc.subcore_barrier()
    for t in range(RPT // ZR):
        r0 = sid * RPT + t * ZR
        pltpu.sync_copy(acc.at[pl.ds(r0, ZR)], out_hbm.at[cid, pl.ds(r0, ZR)])


@functools.partial(
    pl.kernel,
    out_type=jax.ShapeDtypeStruct((NC, NP, D), jnp.float32),
    mesh=_mesh,
    scratch_types=[
        pltpu.VMEM((SR, B), jnp.int32),
        pltpu.VMEM((SR, B), jnp.int32),
        pltpu.VMEM((B, D), jnp.float32),
        pltpu.VMEM((B, D), jnp.float32),
        pltpu.VMEM_SHARED((NP, D), jnp.float32),
        pltpu.SemaphoreType.DMA,
        pltpu.SemaphoreType.DMA,
        pltpu.SemaphoreType.DMA,
        pltpu.SemaphoreType.DMA,
    ],
)
def _sc_aggregate(z_hbm, src2_hbm, dst2_hbm, out_hbm,
                  sidxv, didxv, rows0, rows1, acc, gsem0, gsem1, ssem0, ssem1):
    cid = lax.axis_index("c")
    sid = lax.axis_index("s")
    wid = sid * NC + cid

    _zero_fill(rows0, B)
    for t in range(RPT // ZR):
        pltpu.sync_copy(rows0, acc.at[pl.ds(sid * RPT + t * ZR, ZR)])
    plsc.subcore_barrier()

    rows = (rows0, rows1)
    gsems = (gsem0, gsem1)
    ssems = (ssem0, ssem1)
    gcnt = (SLOTS + NW - 1 - wid) // NW

    def body(s, _):
        @pl.when(s < gcnt)
        def _():
            brow = (wid + NW * s) * SR
            pltpu.sync_copy(src2_hbm.at[pl.ds(brow, SR)], sidxv)
            pltpu.sync_copy(dst2_hbm.at[pl.ds(brow, SR)], didxv)
            hg = [None, None]
            hs = [None, None]
            hg[0] = pltpu.async_copy(z_hbm.at[sidxv.at[0]], rows[0], gsems[0])
            for r in range(SR):
                p = r % 2
                hg[p].wait()
                hs[p] = pltpu.async_copy(rows[p], acc.at[didxv.at[r]], ssems[p], add=True)
                if r + 1 < SR:
                    if hs[1 - p] is not None:
                        hs[1 - p].wait()
                    hg[1 - p] = pltpu.async_copy(
                        z_hbm.at[sidxv.at[r + 1]], rows[1 - p], gsems[1 - p])
            hs[0].wait()
            hs[1].wait()
        return 0

    lax.fori_loop(0, (SLOTS + NW - 1) // NW, body, 0)

    @pl.when(wid == NW - 1)
    def _():
        pltpu.sync_copy(src2_hbm.at[pl.ds(SLOTS * SR, TAILR)], sidxv.at[pl.ds(0, TAILR)])
        pltpu.sync_copy(dst2_hbm.at[pl.ds(SLOTS * SR, TAILR)], didxv.at[pl.ds(0, TAILR)])
        for r in range(TAILR):
            pltpu.async_copy(z_hbm.at[sidxv.at[r]], rows0, gsem0).wait()
            pltpu.sync_copy(rows0, acc.at[didxv.at[r]], add=True)

    plsc.subcore_barrier()
    for t in range(RPT // ZR):
        r0 = sid * RPT + t * ZR
        pltpu.sync_copy(acc.at[pl.ds(r0, ZR)], out_hbm.at[cid, pl.ds(r0, ZR)])


R = 10000


def _deg_col(p_ref):
    return lax.rsqrt(jnp.maximum(p_ref[0, :, 0:1], 1.0))


def _tc_scale_body(x_ref, odp_ref, o_ref):
    o_ref[...] = x_ref[...] * _deg_col(odp_ref)


_tc_scale = pl.pallas_call(
    _tc_scale_body,
    grid=(N // R,),
    in_specs=[
        pl.BlockSpec((R, D), lambda i: (i, 0)),
        pl.BlockSpec((1, R, D), lambda i: (0, i, 0)),
    ],
    out_specs=pl.BlockSpec((R, D), lambda i: (i, 0)),
    out_shape=jax.ShapeDtypeStruct((N, D), jnp.float32),
)


def _tc_layer1_body(a_ref, idp_ref, odp_ref, w1_ref, b1_ref, w2_ref, o_ref):
    a = (a_ref[0] + a_ref[1]) * _deg_col(idp_ref)
    h = jnp.dot(a, w1_ref[...], preferred_element_type=jnp.float32) + b1_ref[...]
    h = jnp.maximum(h, 0.0) * _deg_col(odp_ref)
    o_ref[...] = jnp.dot(h, w2_ref[...], preferred_element_type=jnp.float32)


_tc_layer1 = pl.pallas_call(
    _tc_layer1_body,
    grid=(N // R,),
    in_specs=[
        pl.BlockSpec((NC, R, D), lambda i: (0, i, 0)),
        pl.BlockSpec((1, R, D), lambda i: (1, i, 0)),
        pl.BlockSpec((1, R, D), lambda i: (0, i, 0)),
        pl.BlockSpec((128, 256), lambda i: (0, 0)),
        pl.BlockSpec((1, 256), lambda i: (0, 0)),
        pl.BlockSpec((256, 128), lambda i: (0, 0)),
    ],
    out_specs=pl.BlockSpec((R, D), lambda i: (i, 0)),
    out_shape=jax.ShapeDtypeStruct((N, D), jnp.float32),
)


def _tc_out_body(a_ref, idp_ref, b2_ref, wl_ref, bl_ref, o_ref):
    h = (a_ref[0] + a_ref[1]) * _deg_col(idp_ref) + b2_ref[...]
    h = jnp.maximum(h, 0.0)
    logits = jnp.dot(h, wl_ref[...], preferred_element_type=jnp.float32) + bl_ref[...]
    m = jnp.max(logits, axis=-1, keepdims=True)
    e = jnp.exp(logits - m)
    o_ref[...] = e / jnp.sum(e, axis=-1, keepdims=True)


_tc_out = pl.pallas_call(
    _tc_out_body,
    grid=(N // R,),
    in_specs=[
        pl.BlockSpec((NC, R, D), lambda i: (0, i, 0)),
        pl.BlockSpec((1, R, D), lambda i: (1, i, 0)),
        pl.BlockSpec((1, 128), lambda i: (0, 0)),
        pl.BlockSpec((128, 40), lambda i: (0, 0)),
        pl.BlockSpec((1, 40), lambda i: (0, 0)),
    ],
    out_specs=pl.BlockSpec((R, 40), lambda i: (i, 0)),
    out_shape=jax.ShapeDtypeStruct((N, 40), jnp.float32),
)


def kernel(x, edge_index, W1, b1, W2, b2, Wl, bl):
    src2 = edge_index[0].reshape(ER, B)
    dst2 = edge_index[1].reshape(ER, B)
    deg = _sc_count2(src2, dst2)
    x1 = _tc_scale(x, deg)
    a1p = _sc_aggregate(x1, src2, dst2)
    t2 = _tc_layer1(a1p, deg, deg, W1, b1.reshape(1, -1), W2)
    a2p = _sc_aggregate(t2, src2, dst2)
    return _tc_out(a2p, deg, b2.reshape(1, -1), Wl, bl.reshape(1, -1))


# speedup vs baseline: 8.0015x; 1.0372x over previous
import functools

import jax
import jax.numpy as jnp
from jax import lax
from jax.experimental import pallas as pl
from jax.experimental.pallas import tpu as pltpu
from jax.experimental.pallas import tpu_sc as plsc

N = 10000
NP = 10240
E = 320000
D = 128
B = 128
ER = E // B
SR = 16
SLOTS = ER // SR
TAILR = ER - SLOTS * SR
CSR = 16
CSLOTS = ER // CSR
CTAILR = ER - CSLOTS * CSR
NC = 2
NS = 16
NW = NC * NS
RPT = NP // NS
ZR = 128

_mesh = plsc.VectorSubcoreMesh(core_axis_name="c", subcore_axis_name="s")


def _zero_fill(buf, nrows):
    def fill(i, _):
        for c8 in range(D // 16):
            buf[i, pl.ds(c8 * 16, 16)] = jnp.zeros((16,), jnp.float32)
        return 0

    lax.fori_loop(0, nrows, fill, 0)


@functools.partial(
    pl.kernel,
    out_type=jax.ShapeDtypeStruct((NC, NP, D), jnp.float32),
    mesh=_mesh,
    scratch_types=[
        pltpu.VMEM((CSR, B), jnp.int32),
        pltpu.VMEM((B, D), jnp.float32),
        pltpu.VMEM_SHARED((NP, D), jnp.float32),
        pltpu.SemaphoreType.DMA,
    ],
)
def _sc_count2(src2_hbm, dst2_hbm, out_hbm, idxv, ones, acc, sem):
    cid = lax.axis_index("c")
    sid = lax.axis_index("s")

    _zero_fill(ones, B)
    for t in range(RPT // ZR):
        pltpu.sync_copy(ones, acc.at[pl.ds(sid * RPT + t * ZR, ZR)])

    def fill1(i, _):
        for c8 in range(D // 16):
            ones[i, pl.ds(c8 * 16, 16)] = jnp.ones((16,), jnp.float32)
        return 0

    lax.fori_loop(0, B, fill1, 0)
    plsc.subcore_barrier()

    gcnt = (CSLOTS + NS - 1 - sid) // NS

    def make_body(edges2):
        def body(s, _):
            @pl.when(s < gcnt)
            def _():
                brow = (sid + NS * s) * CSR
                pltpu.sync_copy(edges2.at[pl.ds(brow, CSR)], idxv)
                hs = []
                for r in range(CSR):
                    hs.append(pltpu.async_copy(ones, acc.at[idxv.at[r]], sem, add=True))
                for h in hs:
                    h.wait()
            return 0

        return body

    def tail(edges2):
        pltpu.sync_copy(edges2.at[pl.ds(CSLOTS * CSR, CTAILR)], idxv.at[pl.ds(0, CTAILR)])
        for r in range(CTAILR):
            pltpu.sync_copy(ones, acc.at[idxv.at[r]], add=True)

    @pl.when(cid == 0)
    def _():
        lax.fori_loop(0, (CSLOTS + NS - 1) // NS, make_body(src2_hbm), 0)

        @pl.when(sid == NS - 1)
        def _():
            tail(src2_hbm)

    @pl.when(cid == 1)
    def _():
        lax.fori_loop(0, (CSLOTS + NS - 1) // NS, make_body(dst2_hbm), 0)

        @pl.when(sid == NS - 1)
        def _():
            tail(dst2_hbm)

    plsc.subcore_barrier()
    for t in range(RPT // ZR):
        r0 = sid * RPT + t * ZR
        pltpu.sync_copy(acc.at[pl.ds(r0, ZR)], out_hbm.at[cid, pl.ds(r0, ZR)])


@functools.partial(
    pl.kernel,
    out_type=jax.ShapeDtypeStruct((NC, NP, D), jnp.float32),
    mesh=_mesh,
    scratch_types=[
        pltpu.VMEM((SR, B), jnp.int32),
        pltpu.VMEM((SR, B), jnp.int32),
        pltpu.VMEM((B, D), jnp.float32),
        pltpu.VMEM((B, D), jnp.float32),
        pltpu.VMEM_SHARED((NP, D), jnp.float32),
        pltpu.SemaphoreType.DMA,
        pltpu.SemaphoreType.DMA,
        pltpu.SemaphoreType.DMA,
        pltpu.SemaphoreType.DMA,
    ],
)
def _sc_aggregate(z_hbm, src2_hbm, dst2_hbm, out_hbm,
                  sidxv, didxv, rows0, rows1, acc, gsem0, gsem1, ssem0, ssem1):
    cid = lax.axis_index("c")
    sid = lax.axis_index("s")
    wid = sid * NC + cid

    _zero_fill(rows0, B)
    for t in range(RPT // ZR):
        pltpu.sync_copy(rows0, acc.at[pl.ds(sid * RPT + t * ZR, ZR)])
    plsc.subcore_barrier()

    rows = (rows0, rows1)
    gsems = (gsem0, gsem1)
    ssems = (ssem0, ssem1)
    gcnt = (SLOTS + NW - 1 - wid) // NW

    def body(s, _):
        @pl.when(s < gcnt)
        def _():
            brow = (wid + NW * s) * SR
            pltpu.sync_copy(src2_hbm.at[pl.ds(brow, SR)], sidxv)
            pltpu.sync_copy(dst2_hbm.at[pl.ds(brow, SR)], didxv)
            hg = [None, None]
            hs = [None, None]
            hg[0] = pltpu.async_copy(z_hbm.at[sidxv.at[0]], rows[0], gsems[0])
            for r in range(SR):
                p = r % 2
                hg[p].wait()
                hs[p] = pltpu.async_copy(rows[p], acc.at[didxv.at[r]], ssems[p], add=True)
                if r + 1 < SR:
                    if hs[1 - p] is not None:
                        hs[1 - p].wait()
                    hg[1 - p] = pltpu.async_copy(
                        z_hbm.at[sidxv.at[r + 1]], rows[1 - p], gsems[1 - p])
            hs[0].wait()
            hs[1].wait()
        return 0

    lax.fori_loop(0, (SLOTS + NW - 1) // NW, body, 0)

    @pl.when(wid == NW - 1)
    def _():
        pltpu.sync_copy(src2_hbm.at[pl.ds(SLOTS * SR, TAILR)], sidxv.at[pl.ds(0, TAILR)])
        pltpu.sync_copy(dst2_hbm.at[pl.ds(SLOTS * SR, TAILR)], didxv.at[pl.ds(0, TAILR)])
        for r in range(TAILR):
            pltpu.async_copy(z_hbm.at[sidxv.at[r]], rows0, gsem0).wait()
            pltpu.sync_copy(rows0, acc.at[didxv.at[r]], add=True)

    plsc.subcore_barrier()
    for t in range(RPT // ZR):
        r0 = sid * RPT + t * ZR
        pltpu.sync_copy(acc.at[pl.ds(r0, ZR)], out_hbm.at[cid, pl.ds(r0, ZR)])


R = 10000


def _deg_col(p_ref):
    return lax.rsqrt(jnp.maximum(p_ref[0, :, 0:1], 1.0))


def _tc_scale_body(x_ref, odp_ref, o_ref):
    o_ref[...] = x_ref[...] * _deg_col(odp_ref)


_tc_scale = pl.pallas_call(
    _tc_scale_body,
    grid=(N // R,),
    in_specs=[
        pl.BlockSpec((R, D), lambda i: (i, 0)),
        pl.BlockSpec((1, R, D), lambda i: (0, i, 0)),
    ],
    out_specs=pl.BlockSpec((R, D), lambda i: (i, 0)),
    out_shape=jax.ShapeDtypeStruct((N, D), jnp.float32),
)


def _tc_layer1_body(a_ref, idp_ref, odp_ref, w1_ref, b1_ref, w2_ref, o_ref):
    a = (a_ref[0] + a_ref[1]) * _deg_col(idp_ref)
    h = jnp.dot(a, w1_ref[...], preferred_element_type=jnp.float32) + b1_ref[...]
    h = jnp.maximum(h, 0.0) * _deg_col(odp_ref)
    o_ref[...] = jnp.dot(h, w2_ref[...], preferred_element_type=jnp.float32)


_tc_layer1 = pl.pallas_call(
    _tc_layer1_body,
    grid=(N // R,),
    in_specs=[
        pl.BlockSpec((NC, R, D), lambda i: (0, i, 0)),
        pl.BlockSpec((1, R, D), lambda i: (1, i, 0)),
        pl.BlockSpec((1, R, D), lambda i: (0, i, 0)),
        pl.BlockSpec((128, 256), lambda i: (0, 0)),
        pl.BlockSpec((1, 256), lambda i: (0, 0)),
        pl.BlockSpec((256, 128), lambda i: (0, 0)),
    ],
    out_specs=pl.BlockSpec((R, D), lambda i: (i, 0)),
    out_shape=jax.ShapeDtypeStruct((N, D), jnp.float32),
)


def _tc_out_body(a_ref, idp_ref, b2_ref, wl_ref, bl_ref, o_ref):
    h = (a_ref[0] + a_ref[1]) * _deg_col(idp_ref) + b2_ref[...]
    h = jnp.maximum(h, 0.0)
    logits = jnp.dot(h, wl_ref[...], preferred_element_type=jnp.float32) + bl_ref[...]
    m = jnp.max(logits, axis=-1, keepdims=True)
    e = jnp.exp(logits - m)
    o_ref[...] = e / jnp.sum(e, axis=-1, keepdims=True)


_tc_out = pl.pallas_call(
    _tc_out_body,
    grid=(N // R,),
    in_specs=[
        pl.BlockSpec((NC, R, D), lambda i: (0, i, 0)),
        pl.BlockSpec((1, R, D), lambda i: (1, i, 0)),
        pl.BlockSpec((1, 128), lambda i: (0, 0)),
        pl.BlockSpec((128, 40), lambda i: (0, 0)),
        pl.BlockSpec((1, 40), lambda i: (0, 0)),
    ],
    out_specs=pl.BlockSpec((R, 40), lambda i: (i, 0)),
    out_shape=jax.ShapeDtypeStruct((N, 40), jnp.float32),
)


def kernel(x, edge_index, W1, b1, W2, b2, Wl, bl):
    src2 = edge_index[0].reshape(ER, B)
    dst2 = edge_index[1].reshape(ER, B)
    deg = _sc_count2(src2, dst2)
    x1 = _tc_scale(x, deg)
    a1p = _sc_aggregate(x1, src2, dst2)
    t2 = _tc_layer1(a1p, deg, deg, W1, b1.reshape(1, -1), W2)
    a2p = _sc_aggregate(t2, src2, dst2)
    return _tc_out(a2p, deg, b2.reshape(1, -1), Wl, bl.reshape(1, -1))
